# baseline (speedup 1.0000x reference)
import jax
import jax.numpy as jnp
from jax import lax
from jax.experimental import pallas as pl
from jax.experimental.pallas import tpu as pltpu
from jax.experimental.pallas import tpu_sc as plsc

_B = 4096
_F = 26
_V = 100000
_FP = 32
_NC = 2
_NS = 16
_FH = _FP // _NC
_BPT = _B // _NS
_LANES = 16
_VS = 100096
_CW = 12544
_CWL = _VS - 7 * _CW


def _sc_body(idx_hbm, w_hbm, bias_hbm, part_hbm, scratch_hbm,
             bounce, idx_v, gath_v, bias_v, out_v, sem):
    sc = lax.axis_index("c")
    s = lax.axis_index("s")

    g = s // 8
    k = s % 8
    rows0 = pl.multiple_of(sc * _FH + g * 8, 8)
    c0 = pl.multiple_of(k * _CW, 128)

    def _stage(size):
        pltpu.sync_copy(w_hbm.at[pl.ds(rows0, 8), pl.ds(c0, size)],
                        bounce.at[:, pl.ds(0, size)])
        for r in range(8):
            pltpu.sync_copy(
                bounce.at[r, pl.ds(0, size)],
                scratch_hbm.at[pl.ds((rows0 + r) * _VS + c0, size)])

    @pl.when(k < 7)
    def _():
        _stage(_CW)

    @pl.when(k == 7)
    def _():
        _stage(_CWL)

    pltpu.sync_copy(
        idx_hbm.at[pl.ds(pl.multiple_of(sc * _FH, 8), _FH),
                   pl.ds(pl.multiple_of(s * _BPT, 128), _BPT)],
        idx_v)
    pltpu.sync_copy(bias_hbm, bias_v)
    plsc.subcore_barrier()

    row_base = sc * _FH
    for fl in range(_FH):
        off = (row_base + fl) * _VS
        for c in range(_BPT // _LANES):
            sl = pl.ds(c * _LANES, _LANES)
            idx_v[fl, sl] = idx_v[fl, sl] + off

    cps = [
        pltpu.make_async_copy(
            scratch_hbm.at[idx_v.at[fl, pl.ds(h * 128, 128)]],
            gath_v.at[fl, pl.ds(h * 128, 128)], sem)
        for fl in range(_FH) for h in range(2)
    ]
    for cp in cps:
        cp.start()
    for cp in cps:
        cp.wait()

    bias_eff = jnp.where(sc == 0, bias_v[...], jnp.zeros((_LANES,), jnp.float32))
    for c in range(_BPT // _LANES):
        sl = pl.ds(c * _LANES, _LANES)
        acc = gath_v[0, sl] + bias_eff
        for fl in range(1, _FH):
            acc = acc + gath_v[fl, sl]
        out_v[sl] = acc

    pltpu.sync_copy(out_v, part_hbm.at[pl.ds(sc * _B + s * _BPT, _BPT)])


@jax.jit
def kernel(indices, weights, bias):
    idx_t = jnp.pad(indices.astype(jnp.int32).T, ((0, _FP - _F), (0, 0)))
    w2 = jnp.pad(weights.reshape(_F, _V), ((0, _FP - _F), (0, _VS - _V)))
    bias16 = jnp.broadcast_to(bias.reshape(1), (_LANES,)).astype(jnp.float32)

    mesh = plsc.VectorSubcoreMesh(
        core_axis_name="c", subcore_axis_name="s",
        num_cores=_NC, num_subcores=_NS)

    partials, _ = pl.kernel(
        _sc_body,
        out_type=(jax.ShapeDtypeStruct((_NC * _B,), jnp.float32),
                  jax.ShapeDtypeStruct((_FP * _VS,), jnp.float32)),
        mesh=mesh,
        scratch_types=[
            pltpu.VMEM((8, _CW), jnp.float32),
            pltpu.VMEM((_FH, _BPT), jnp.int32),
            pltpu.VMEM((_FH, _BPT), jnp.float32),
            pltpu.VMEM((_LANES,), jnp.float32),
            pltpu.VMEM((_BPT,), jnp.float32),
            pltpu.SemaphoreType.DMA,
        ],
    )(idx_t, w2, bias16)

    return (partials[:_B] + partials[_B:]).reshape(_B, 1)


# speedup vs baseline: 1.7270x; 1.7270x over previous
import jax
import jax.numpy as jnp
from jax import lax
from jax.experimental import pallas as pl
from jax.experimental.pallas import tpu as pltpu
from jax.experimental.pallas import tpu_sc as plsc

_B = 4096
_F = 26
_V = 100000
_FP = 32
_NC = 2
_NS = 16
_FH = _FP // _NC
_F1 = _F - _FH
_BPT = _B // _NS
_LANES = 16
_VS = 100096
_CW = 12544
_CWL = _VS - 7 * _CW


def _gather_reduce(scratch_hbm, idx_v, gath_v, out_v, sem, nf, row_base,
                   bias_eff):
    for fl in range(nf):
        off = (row_base + fl) * _VS
        for c in range(_BPT // _LANES):
            sl = pl.ds(c * _LANES, _LANES)
            idx_v[fl, sl] = idx_v[fl, sl] + off

    cps = [
        pltpu.make_async_copy(
            scratch_hbm.at[idx_v.at[fl, pl.ds(h * 128, 128)]],
            gath_v.at[fl, pl.ds(h * 128, 128)], sem)
        for fl in range(nf) for h in range(2)
    ]
    for cp in cps:
        cp.start()
    for cp in cps:
        cp.wait()

    for c in range(_BPT // _LANES):
        sl = pl.ds(c * _LANES, _LANES)
        acc = gath_v[0, sl] + bias_eff
        for fl in range(1, nf):
            acc = acc + gath_v[fl, sl]
        out_v[sl] = acc


def _sc_body(idx_hbm, tail_hbm, w_hbm, bias_hbm, part_hbm, scratch_hbm,
             bounce, idx_v, gath_v, bias_v, out_v, sem):
    sc = lax.axis_index("c")
    s = lax.axis_index("s")

    g = s // 8
    k = s % 8
    rows0 = pl.multiple_of(sc * _FH + g * 8, 8)
    c0 = pl.multiple_of(k * _CW, 128)

    def _stage(size):
        pltpu.sync_copy(w_hbm.at[pl.ds(rows0, 8), pl.ds(c0, size)],
                        bounce.at[:, pl.ds(0, size)])
        for r in range(8):
            pltpu.sync_copy(
                bounce.at[r, pl.ds(0, size)],
                scratch_hbm.at[pl.ds((rows0 + r) * _VS + c0, size)])

    @pl.when(k < 7)
    def _():
        _stage(_CW)

    @pl.when(k == 7)
    def _():
        _stage(_CWL)

    col = pl.ds(pl.multiple_of(s * _BPT, 128), _BPT)

    @pl.when(sc == 0)
    def _():
        pltpu.sync_copy(idx_hbm.at[pl.ds(0, _FH), col], idx_v)

    @pl.when(sc == 1)
    def _():
        pltpu.sync_copy(idx_hbm.at[pl.ds(16, 8), col],
                        idx_v.at[pl.ds(0, 8)])
        pltpu.sync_copy(tail_hbm.at[:, col], idx_v.at[pl.ds(8, 2)])

    pltpu.sync_copy(bias_hbm, bias_v)
    plsc.subcore_barrier()

    bias_eff = jnp.where(sc == 0, bias_v[...], jnp.zeros((_LANES,), jnp.float32))

    @pl.when(sc == 0)
    def _():
        _gather_reduce(scratch_hbm, idx_v, gath_v, out_v, sem, _FH, 0,
                       bias_eff)

    @pl.when(sc == 1)
    def _():
        _gather_reduce(scratch_hbm, idx_v, gath_v, out_v, sem, _F1, _FH,
                       bias_eff)

    pltpu.sync_copy(out_v, part_hbm.at[pl.ds(sc * _B + s * _BPT, _BPT)])


@jax.jit
def kernel(indices, weights, bias):
    idx_t = indices.astype(jnp.int32).T
    idx_tail = lax.slice(idx_t, (24, 0), (26, _B))
    w2 = jnp.pad(weights.reshape(_F, _V), ((0, _FP - _F), (0, _VS - _V)))
    bias16 = jnp.broadcast_to(bias.reshape(1), (_LANES,)).astype(jnp.float32)

    mesh = plsc.VectorSubcoreMesh(
        core_axis_name="c", subcore_axis_name="s",
        num_cores=_NC, num_subcores=_NS)

    partials, _ = pl.kernel(
        _sc_body,
        out_type=(jax.ShapeDtypeStruct((_NC * _B,), jnp.float32),
                  jax.ShapeDtypeStruct((_FP * _VS,), jnp.float32)),
        mesh=mesh,
        scratch_types=[
            pltpu.VMEM((8, _CW), jnp.float32),
            pltpu.VMEM((_FH, _BPT), jnp.int32),
            pltpu.VMEM((_FH, _BPT), jnp.float32),
            pltpu.VMEM((_LANES,), jnp.float32),
            pltpu.VMEM((_BPT,), jnp.float32),
            pltpu.SemaphoreType.DMA,
        ],
    )(idx_t, idx_tail, w2, bias16)

    return (partials[:_B] + partials[_B:]).reshape(_B, 1)
